# baseline (device time: 568139 ns/iter reference)
import jax
import jax.numpy as jnp
from jax import lax
from jax.experimental import pallas as pl
from jax.experimental.pallas import tpu as pltpu

K = 8


def kernel(x):
    x = x.astype(jnp.bfloat16)
    m, n = x.shape
    n_half = n // 2
    r = m // K

    def body(x_ref, out_ref, vbuf, local_sem, pack_sems, send_sems, recv_sems):
        my_x = lax.axis_index("x")
        my_y = lax.axis_index("y")
        nbr = (my_x, 1 - my_y)

        barrier = pltpu.get_barrier_semaphore()
        pl.semaphore_signal(
            barrier, inc=1, device_id=nbr, device_id_type=pl.DeviceIdType.MESH
        )
        pl.semaphore_wait(barrier, 1)

        local = pltpu.make_async_copy(
            x_ref.at[:, pl.ds(my_y * n_half, n_half)],
            out_ref.at[pl.ds(my_y * m, m), :],
            local_sem,
        )
        local.start()

        packs = []
        for k in range(K):
            p = pltpu.make_async_copy(
                x_ref.at[pl.ds(k * r, r), pl.ds((1 - my_y) * n_half, n_half)],
                vbuf.at[k],
                pack_sems.at[k],
            )
            p.start()
            packs.append(p)

        rdmas = []
        for k in range(K):
            packs[k].wait()
            rd = pltpu.make_async_remote_copy(
                src_ref=vbuf.at[k],
                dst_ref=out_ref.at[pl.ds(my_y * m + k * r, r), :],
                send_sem=send_sems.at[k],
                recv_sem=recv_sems.at[k],
                device_id=nbr,
                device_id_type=pl.DeviceIdType.MESH,
            )
            rd.start()
            rdmas.append(rd)

        local.wait()
        for rd in rdmas:
            rd.wait_send()

        for k in range(K):
            recv = pltpu.make_async_remote_copy(
                src_ref=vbuf.at[k],
                dst_ref=out_ref.at[pl.ds((1 - my_y) * m + k * r, r), :],
                send_sem=send_sems.at[k],
                recv_sem=recv_sems.at[k],
                device_id=nbr,
                device_id_type=pl.DeviceIdType.MESH,
            )
            recv.wait_recv()

    return pl.pallas_call(
        body,
        out_shape=jax.ShapeDtypeStruct((2 * m, n_half), jnp.bfloat16),
        in_specs=[pl.BlockSpec(memory_space=pltpu.MemorySpace.HBM)],
        out_specs=pl.BlockSpec(memory_space=pltpu.MemorySpace.HBM),
        scratch_shapes=[
            pltpu.VMEM((K, r, n_half), jnp.bfloat16),
            pltpu.SemaphoreType.DMA,
            pltpu.SemaphoreType.DMA((K,)),
            pltpu.SemaphoreType.DMA((K,)),
            pltpu.SemaphoreType.DMA((K,)),
        ],
        compiler_params=pltpu.CompilerParams(collective_id=0),
    )(x)


# device time: 164369 ns/iter; 3.4565x vs baseline; 3.4565x over previous
import jax
import jax.numpy as jnp
from jax import lax
from jax.experimental import pallas as pl
from jax.experimental.pallas import tpu as pltpu

KY = 16
KL = 16


def kernel(x):
    x = x.astype(jnp.bfloat16)
    m, n = x.shape
    n_half = n // 2
    h = m // 2
    r = h // KY
    rl = m // KL

    def body(
        x_ref, out_ref,
        ybuf, rbuf, lbuf,
        ypack_sems, ysend_sems, yrecv_sems,
        xsend_sems, xrecv_sems,
        store_sems, lpack_sems, lout_sems,
    ):
        my_x = lax.axis_index("x")
        my_y = lax.axis_index("y")
        y_nbr = (my_x, 1 - my_y)
        x_nbr = (1 - my_x, my_y)

        barrier = pltpu.get_barrier_semaphore()
        for nbr in (y_nbr, x_nbr):
            pl.semaphore_signal(
                barrier, inc=1, device_id=nbr,
                device_id_type=pl.DeviceIdType.MESH,
            )
        pl.semaphore_wait(barrier, 2)

        ypacks = []
        for k in range(KY):
            p = pltpu.make_async_copy(
                x_ref.at[
                    pl.ds(my_x * h + k * r, r),
                    pl.ds((1 - my_y) * n_half, n_half),
                ],
                ybuf.at[k],
                ypack_sems.at[k],
            )
            p.start()
            ypacks.append(p)

        y_rdmas = []
        for k in range(KY):
            ypacks[k].wait()
            rd = pltpu.make_async_remote_copy(
                src_ref=ybuf.at[k],
                dst_ref=rbuf.at[k],
                send_sem=ysend_sems.at[k],
                recv_sem=yrecv_sems.at[k],
                device_id=y_nbr,
                device_id_type=pl.DeviceIdType.MESH,
            )
            rd.start()
            y_rdmas.append(rd)

        lpacks = []
        for j in range(KL):
            p = pltpu.make_async_copy(
                x_ref.at[pl.ds(j * rl, rl), pl.ds(my_y * n_half, n_half)],
                lbuf.at[j],
                lpack_sems.at[j],
            )
            p.start()
            lpacks.append(p)
        louts = []
        for j in range(KL):
            lpacks[j].wait()
            o = pltpu.make_async_copy(
                lbuf.at[j],
                out_ref.at[pl.ds(my_y * m + j * rl, rl), :],
                lout_sems.at[j],
            )
            o.start()
            louts.append(o)

        stores = []
        x_rdmas = []
        for k in range(KY):
            recv = pltpu.make_async_remote_copy(
                src_ref=ybuf.at[k],
                dst_ref=rbuf.at[k],
                send_sem=ysend_sems.at[k],
                recv_sem=yrecv_sems.at[k],
                device_id=y_nbr,
                device_id_type=pl.DeviceIdType.MESH,
            )
            recv.wait_recv()
            rows = (1 - my_y) * m + my_x * h + k * r
            st = pltpu.make_async_copy(
                rbuf.at[k],
                out_ref.at[pl.ds(rows, r), :],
                store_sems.at[k],
            )
            st.start()
            stores.append(st)
            fw = pltpu.make_async_remote_copy(
                src_ref=rbuf.at[k],
                dst_ref=out_ref.at[pl.ds(rows, r), :],
                send_sem=xsend_sems.at[k],
                recv_sem=xrecv_sems.at[k],
                device_id=x_nbr,
                device_id_type=pl.DeviceIdType.MESH,
            )
            fw.start()
            x_rdmas.append(fw)

        for rd in y_rdmas:
            rd.wait_send()
        for fw in x_rdmas:
            fw.wait_send()
        for st in stores:
            st.wait()
        for o in louts:
            o.wait()
        for k in range(KY):
            rows = (1 - my_y) * m + (1 - my_x) * h + k * r
            recv = pltpu.make_async_remote_copy(
                src_ref=rbuf.at[k],
                dst_ref=out_ref.at[pl.ds(rows, r), :],
                send_sem=xsend_sems.at[k],
                recv_sem=xrecv_sems.at[k],
                device_id=x_nbr,
                device_id_type=pl.DeviceIdType.MESH,
            )
            recv.wait_recv()

    return pl.pallas_call(
        body,
        out_shape=jax.ShapeDtypeStruct((2 * m, n_half), jnp.bfloat16),
        in_specs=[pl.BlockSpec(memory_space=pltpu.MemorySpace.HBM)],
        out_specs=pl.BlockSpec(memory_space=pltpu.MemorySpace.HBM),
        scratch_shapes=[
            pltpu.VMEM((KY, h // KY, n_half), jnp.bfloat16),
            pltpu.VMEM((KY, h // KY, n_half), jnp.bfloat16),
            pltpu.VMEM((KL, m // KL, n_half), jnp.bfloat16),
            pltpu.SemaphoreType.DMA((KY,)),
            pltpu.SemaphoreType.DMA((KY,)),
            pltpu.SemaphoreType.DMA((KY,)),
            pltpu.SemaphoreType.DMA((KY,)),
            pltpu.SemaphoreType.DMA((KY,)),
            pltpu.SemaphoreType.DMA((KY,)),
            pltpu.SemaphoreType.DMA((KL,)),
            pltpu.SemaphoreType.DMA((KL,)),
        ],
        compiler_params=pltpu.CompilerParams(collective_id=0),
    )(x)


# device time: 139543 ns/iter; 4.0714x vs baseline; 1.1779x over previous
import jax
import jax.numpy as jnp
from jax import lax
from jax.experimental import pallas as pl
from jax.experimental.pallas import tpu as pltpu

KY = 16
KL = 16
NF = 4


def kernel(x):
    m, n = x.shape
    n_half = n // 2
    h = m // 2
    r = h // KY
    rl = m // KL

    def body(
        x_ref, out_ref,
        yf32, lf32, ybuf, rbuf, lbuf,
        ypack_sems, ysend_sems, yrecv_sems,
        xsend_sems, xrecv_sems,
        store_sems, lpack_sems, lout_sems,
    ):
        my_x = lax.axis_index("x")
        my_y = lax.axis_index("y")
        y_nbr = (my_x, 1 - my_y)
        x_nbr = (1 - my_x, my_y)

        barrier = pltpu.get_barrier_semaphore()
        for nbr in (y_nbr, x_nbr):
            pl.semaphore_signal(
                barrier, inc=1, device_id=nbr,
                device_id_type=pl.DeviceIdType.MESH,
            )
        pl.semaphore_wait(barrier, 2)

        def ypack(k):
            p = pltpu.make_async_copy(
                x_ref.at[
                    pl.ds(my_x * h + k * r, r),
                    pl.ds((1 - my_y) * n_half, n_half),
                ],
                yf32.at[k % NF],
                ypack_sems.at[k % NF],
            )
            p.start()
            return p

        def lpack(j):
            p = pltpu.make_async_copy(
                x_ref.at[pl.ds(j * rl, rl), pl.ds(my_y * n_half, n_half)],
                lf32.at[j % NF],
                lpack_sems.at[j % NF],
            )
            p.start()
            return p

        ypacks = {k: ypack(k) for k in range(NF)}
        y_rdmas = []
        for k in range(KY):
            ypacks[k].wait()
            ybuf[k, :, :] = yf32[k % NF, :, :].astype(jnp.bfloat16)
            if k + NF < KY:
                ypacks[k + NF] = ypack(k + NF)
            rd = pltpu.make_async_remote_copy(
                src_ref=ybuf.at[k],
                dst_ref=rbuf.at[k],
                send_sem=ysend_sems.at[k],
                recv_sem=yrecv_sems.at[k],
                device_id=y_nbr,
                device_id_type=pl.DeviceIdType.MESH,
            )
            rd.start()
            y_rdmas.append(rd)

        lpacks = {j: lpack(j) for j in range(NF)}
        louts = []
        for j in range(KL):
            lpacks[j].wait()
            lbuf[j, :, :] = lf32[j % NF, :, :].astype(jnp.bfloat16)
            if j + NF < KL:
                lpacks[j + NF] = lpack(j + NF)
            o = pltpu.make_async_copy(
                lbuf.at[j],
                out_ref.at[pl.ds(my_y * m + j * rl, rl), :],
                lout_sems.at[j],
            )
            o.start()
            louts.append(o)

        stores = []
        x_rdmas = []
        for k in range(KY):
            recv = pltpu.make_async_remote_copy(
                src_ref=ybuf.at[k],
                dst_ref=rbuf.at[k],
                send_sem=ysend_sems.at[k],
                recv_sem=yrecv_sems.at[k],
                device_id=y_nbr,
                device_id_type=pl.DeviceIdType.MESH,
            )
            recv.wait_recv()
            rows = (1 - my_y) * m + my_x * h + k * r
            st = pltpu.make_async_copy(
                rbuf.at[k],
                out_ref.at[pl.ds(rows, r), :],
                store_sems.at[k],
            )
            st.start()
            stores.append(st)
            fw = pltpu.make_async_remote_copy(
                src_ref=rbuf.at[k],
                dst_ref=out_ref.at[pl.ds(rows, r), :],
                send_sem=xsend_sems.at[k],
                recv_sem=xrecv_sems.at[k],
                device_id=x_nbr,
                device_id_type=pl.DeviceIdType.MESH,
            )
            fw.start()
            x_rdmas.append(fw)

        for rd in y_rdmas:
            rd.wait_send()
        for fw in x_rdmas:
            fw.wait_send()
        for st in stores:
            st.wait()
        for o in louts:
            o.wait()
        for k in range(KY):
            rows = (1 - my_y) * m + (1 - my_x) * h + k * r
            recv = pltpu.make_async_remote_copy(
                src_ref=rbuf.at[k],
                dst_ref=out_ref.at[pl.ds(rows, r), :],
                send_sem=xsend_sems.at[k],
                recv_sem=xrecv_sems.at[k],
                device_id=x_nbr,
                device_id_type=pl.DeviceIdType.MESH,
            )
            recv.wait_recv()

    return pl.pallas_call(
        body,
        out_shape=jax.ShapeDtypeStruct((2 * m, n_half), jnp.bfloat16),
        in_specs=[pl.BlockSpec(memory_space=pltpu.MemorySpace.HBM)],
        out_specs=pl.BlockSpec(memory_space=pltpu.MemorySpace.HBM),
        scratch_shapes=[
            pltpu.VMEM((NF, h // KY, n_half), jnp.float32),
            pltpu.VMEM((NF, m // KL, n_half), jnp.float32),
            pltpu.VMEM((KY, h // KY, n_half), jnp.bfloat16),
            pltpu.VMEM((KY, h // KY, n_half), jnp.bfloat16),
            pltpu.VMEM((KL, m // KL, n_half), jnp.bfloat16),
            pltpu.SemaphoreType.DMA((NF,)),
            pltpu.SemaphoreType.DMA((KY,)),
            pltpu.SemaphoreType.DMA((KY,)),
            pltpu.SemaphoreType.DMA((KY,)),
            pltpu.SemaphoreType.DMA((KY,)),
            pltpu.SemaphoreType.DMA((KY,)),
            pltpu.SemaphoreType.DMA((NF,)),
            pltpu.SemaphoreType.DMA((KL,)),
        ],
        compiler_params=pltpu.CompilerParams(
            collective_id=0, vmem_limit_bytes=56 * 1024 * 1024
        ),
    )(x)


# device time: 125321 ns/iter; 4.5335x vs baseline; 1.1135x over previous
import jax
import jax.numpy as jnp
from jax import lax
from jax.experimental import pallas as pl
from jax.experimental.pallas import tpu as pltpu

KY = 16
KL = 16
NF = 4


def kernel(x):
    m, n = x.shape
    n_half = n // 2
    h = m // 2
    r = h // KY
    rl = m // KL

    def body(
        x_ref, out_ref,
        yf32, lf32, ybuf, rbuf, lbuf,
        ypack_sems, ysend_sems, yrecv_sems,
        xsend_sems, xrecv_sems,
        store_sems, lpack_sems, lout_sems,
    ):
        my_x = lax.axis_index("x")
        my_y = lax.axis_index("y")
        y_nbr = (my_x, 1 - my_y)
        x_nbr = (1 - my_x, my_y)

        barrier = pltpu.get_barrier_semaphore()
        for nbr in (y_nbr, x_nbr):
            pl.semaphore_signal(
                barrier, inc=1, device_id=nbr,
                device_id_type=pl.DeviceIdType.MESH,
            )
        pl.semaphore_wait(barrier, 2)

        def ypack(k):
            p = pltpu.make_async_copy(
                x_ref.at[
                    pl.ds(my_x * h + k * r, r),
                    pl.ds((1 - my_y) * n_half, n_half),
                ],
                yf32.at[k % NF],
                ypack_sems.at[k % NF],
            )
            p.start()
            return p

        def lpack(j):
            p = pltpu.make_async_copy(
                x_ref.at[pl.ds(j * rl, rl), pl.ds(my_y * n_half, n_half)],
                lf32.at[j % NF],
                lpack_sems.at[j % NF],
            )
            p.start()
            return p

        ypacks = {k: ypack(k) for k in range(NF)}
        y_rdmas = []
        for k in range(KY):
            ypacks[k].wait()
            ybuf[k, :, :] = yf32[k % NF, :, :].astype(jnp.bfloat16)
            if k + NF < KY:
                ypacks[k + NF] = ypack(k + NF)
            rd = pltpu.make_async_remote_copy(
                src_ref=ybuf.at[k],
                dst_ref=rbuf.at[k],
                send_sem=ysend_sems.at[k],
                recv_sem=yrecv_sems.at[k],
                device_id=y_nbr,
                device_id_type=pl.DeviceIdType.MESH,
            )
            rd.start()
            y_rdmas.append(rd)

        lpacks = {j: lpack(j) for j in range(NF)}
        louts = []
        stores = []
        x_rdmas = []
        for k in range(KY):
            recv = pltpu.make_async_remote_copy(
                src_ref=ybuf.at[k],
                dst_ref=rbuf.at[k],
                send_sem=ysend_sems.at[k],
                recv_sem=yrecv_sems.at[k],
                device_id=y_nbr,
                device_id_type=pl.DeviceIdType.MESH,
            )
            recv.wait_recv()
            rows = (1 - my_y) * m + my_x * h + k * r
            st = pltpu.make_async_copy(
                rbuf.at[k],
                out_ref.at[pl.ds(rows, r), :],
                store_sems.at[k],
            )
            st.start()
            stores.append(st)
            fw = pltpu.make_async_remote_copy(
                src_ref=rbuf.at[k],
                dst_ref=out_ref.at[pl.ds(rows, r), :],
                send_sem=xsend_sems.at[k],
                recv_sem=xrecv_sems.at[k],
                device_id=x_nbr,
                device_id_type=pl.DeviceIdType.MESH,
            )
            fw.start()
            x_rdmas.append(fw)
            if k < KL:
                j = k
                lpacks[j].wait()
                lbuf[j, :, :] = lf32[j % NF, :, :].astype(jnp.bfloat16)
                if j + NF < KL:
                    lpacks[j + NF] = lpack(j + NF)
                o = pltpu.make_async_copy(
                    lbuf.at[j],
                    out_ref.at[pl.ds(my_y * m + j * rl, rl), :],
                    lout_sems.at[j],
                )
                o.start()
                louts.append(o)

        for rd in y_rdmas:
            rd.wait_send()
        for fw in x_rdmas:
            fw.wait_send()
        for st in stores:
            st.wait()
        for o in louts:
            o.wait()
        for k in range(KY):
            rows = (1 - my_y) * m + (1 - my_x) * h + k * r
            recv = pltpu.make_async_remote_copy(
                src_ref=rbuf.at[k],
                dst_ref=out_ref.at[pl.ds(rows, r), :],
                send_sem=xsend_sems.at[k],
                recv_sem=xrecv_sems.at[k],
                device_id=x_nbr,
                device_id_type=pl.DeviceIdType.MESH,
            )
            recv.wait_recv()

    return pl.pallas_call(
        body,
        out_shape=jax.ShapeDtypeStruct((2 * m, n_half), jnp.bfloat16),
        in_specs=[pl.BlockSpec(memory_space=pltpu.MemorySpace.HBM)],
        out_specs=pl.BlockSpec(memory_space=pltpu.MemorySpace.HBM),
        scratch_shapes=[
            pltpu.VMEM((NF, h // KY, n_half), jnp.float32),
            pltpu.VMEM((NF, m // KL, n_half), jnp.float32),
            pltpu.VMEM((KY, h // KY, n_half), jnp.bfloat16),
            pltpu.VMEM((KY, h // KY, n_half), jnp.bfloat16),
            pltpu.VMEM((KL, m // KL, n_half), jnp.bfloat16),
            pltpu.SemaphoreType.DMA((NF,)),
            pltpu.SemaphoreType.DMA((KY,)),
            pltpu.SemaphoreType.DMA((KY,)),
            pltpu.SemaphoreType.DMA((KY,)),
            pltpu.SemaphoreType.DMA((KY,)),
            pltpu.SemaphoreType.DMA((KY,)),
            pltpu.SemaphoreType.DMA((NF,)),
            pltpu.SemaphoreType.DMA((KL,)),
        ],
        compiler_params=pltpu.CompilerParams(
            collective_id=0, vmem_limit_bytes=56 * 1024 * 1024
        ),
    )(x)
